# baseline (speedup 1.0000x reference)
import functools

import jax
import jax.numpy as jnp
from jax import lax
from jax.experimental import pallas as pl
from jax.experimental.pallas import tpu as pltpu
from jax.experimental.pallas import tpu_sc as plsc

_NC, _NS = 2, 16
_NW = _NC * _NS
_LANES = 16
_K = 8
_R = 3


def kernel(x, pos_table):
    B, L, D = x.shape
    tab = pos_table[:L]

    pw = L // _NW
    steps = pw // _K
    nvec = D // _LANES

    mesh = plsc.VectorSubcoreMesh(core_axis_name="c", subcore_axis_name="s")

    @functools.partial(
        pl.kernel,
        out_type=jax.ShapeDtypeStruct((B, L, D), jnp.float32),
        mesh=mesh,
        scratch_types=(
            [
                pltpu.VMEM((_R, _K, D), jnp.float32),
                pltpu.VMEM((_R, B, _K, D), jnp.float32),
            ]
            + [pltpu.SemaphoreType.DMA] * _R
            + [pltpu.SemaphoreType.DMA] * (_R * B)
            + [pltpu.SemaphoreType.DMA] * (_R * B)
        ),
    )
    def sc_add(x_hbm, tab_hbm, out_hbm, tbuf, obuf, *sems):
        sem_t = sems[0:_R]
        sem_x = [sems[_R + r * B:_R + (r + 1) * B] for r in range(_R)]
        sem_o = [sems[_R * (1 + B) + r * B:_R * (1 + B) + (r + 1) * B]
                 for r in range(_R)]

        wid = lax.axis_index("s") * _NC + lax.axis_index("c")
        pos0 = wid * pw

        def issue_ins(j, r):
            prow = pos0 + j * _K
            pltpu.async_copy(tab_hbm.at[pl.ds(prow, _K), :],
                             tbuf.at[r], sem_t[r])
            for b in range(B):
                pltpu.async_copy(x_hbm.at[b, pl.ds(prow, _K), :],
                                 obuf.at[r, b], sem_x[r][b])

        def issue_outs(j, r):
            prow = pos0 + j * _K
            for b in range(B):
                pltpu.async_copy(obuf.at[r, b],
                                 out_hbm.at[b, pl.ds(prow, _K), :],
                                 sem_o[r][b])

        def wait_tab(r):
            pltpu.make_async_copy(tab_hbm.at[pl.ds(0, _K), :],
                                  tbuf.at[r], sem_t[r]).wait()

        def wait_chunk(r, b, sem2):
            pltpu.make_async_copy(x_hbm.at[0, pl.ds(0, _K), :],
                                  obuf.at[r, b], sem2[r][b]).wait()

        def compute(r):
            @plsc.parallel_loop(0, nvec, unroll=2)
            def _(i):
                off = i * _LANES
                for row in range(_K):
                    v = tbuf[r, row, pl.ds(off, _LANES)]
                    for b in range(B):
                        plsc.addupdate(
                            obuf.at[r, b, row, pl.ds(off, _LANES)], v)

        def consume(j, r):
            wait_tab(r)
            for b in range(B):
                wait_chunk(r, b, sem_x)
            compute(r)
            issue_outs(j, r)

        def sub(j, r, out_wait=True, prefetch=True):
            r_next = (r + 1) % _R
            if out_wait:
                for b in range(B):
                    wait_chunk(r_next, b, sem_o)
            if prefetch:
                issue_ins(j + 1, r_next)
            consume(j, r)

        issue_ins(0, 0)
        issue_ins(1, 1)
        consume(0, 0)
        sub(1, 1, out_wait=False)

        def triple(t, _):
            j = 2 + 3 * t
            sub(j, 2)
            sub(j + 1, 0)
            sub(j + 2, 1)
            return _

        lax.fori_loop(0, (steps - 4) // 3, triple, 0)

        sub(steps - 3, (steps - 3) % _R)
        sub(steps - 2, (steps - 2) % _R)
        sub(steps - 1, (steps - 1) % _R, prefetch=False)
        for j in (steps - 2, steps - 1):
            for b in range(B):
                wait_chunk(j % _R, b, sem_o)

    return sc_add(x, tab)


# speedup vs baseline: 1.0228x; 1.0228x over previous
import functools

import jax
import jax.numpy as jnp
from jax import lax
from jax.experimental import pallas as pl
from jax.experimental.pallas import tpu as pltpu
from jax.experimental.pallas import tpu_sc as plsc

_NC, _NS = 2, 16
_NW = _NC * _NS
_LANES = 16
_K = 8


def kernel(x, pos_table):
    B, L, D = x.shape
    tab = pos_table[:L]

    pw = L // _NW
    steps = pw // _K
    nvec = D // _LANES

    mesh = plsc.VectorSubcoreMesh(core_axis_name="c", subcore_axis_name="s")

    @functools.partial(
        pl.kernel,
        out_type=jax.ShapeDtypeStruct((B, L, D), jnp.float32),
        mesh=mesh,
        scratch_types=(
            [
                pltpu.VMEM((2, _K, D), jnp.float32),
                pltpu.VMEM((2, B, _K, D), jnp.float32),
            ]
            + [pltpu.SemaphoreType.DMA] * 2
            + [pltpu.SemaphoreType.DMA] * (2 * B)
            + [pltpu.SemaphoreType.DMA] * (2 * B)
        ),
    )
    def sc_add(x_hbm, tab_hbm, out_hbm, tbuf, obuf, *sems):
        sem_t = sems[0:2]
        sem_x = [sems[2 + p * B:2 + (p + 1) * B] for p in range(2)]
        sem_o = [sems[2 + 2 * B + p * B:2 + 2 * B + (p + 1) * B]
                 for p in range(2)]

        wid = lax.axis_index("s") * _NC + lax.axis_index("c")
        pos0 = wid * pw

        def issue_ins(j, p):
            prow = pos0 + j * _K
            pltpu.async_copy(tab_hbm.at[pl.ds(prow, _K), :],
                             tbuf.at[p], sem_t[p])
            for b in range(B):
                pltpu.async_copy(x_hbm.at[b, pl.ds(prow, _K), :],
                                 obuf.at[p, b], sem_x[p][b])

        def wait_tab(p):
            pltpu.make_async_copy(tab_hbm.at[pl.ds(0, _K), :],
                                  tbuf.at[p], sem_t[p]).wait()

        def wait_chunk(p, b, sem2):
            pltpu.make_async_copy(x_hbm.at[0, pl.ds(0, _K), :],
                                  obuf.at[p, b], sem2[p][b]).wait()

        def compute_b(p, b):
            @plsc.parallel_loop(0, nvec, unroll=2)
            def _(i):
                off = i * _LANES
                for row in range(_K):
                    v = tbuf[p, row, pl.ds(off, _LANES)]
                    plsc.addupdate(obuf.at[p, b, row, pl.ds(off, _LANES)], v)

        def consume(j, p):
            prow = pos0 + j * _K
            wait_tab(p)
            for b in range(B):
                wait_chunk(p, b, sem_x)
                compute_b(p, b)
                pltpu.async_copy(obuf.at[p, b],
                                 out_hbm.at[b, pl.ds(prow, _K), :],
                                 sem_o[p][b])

        def sub(j, p):
            for b in range(B):
                wait_chunk(1 - p, b, sem_o)
            issue_ins(j + 1, 1 - p)
            consume(j, p)

        issue_ins(0, 0)
        issue_ins(1, 1)
        consume(0, 0)

        def pair(j2, _):
            j = 1 + j2 * 2
            sub(j, 1)
            sub(j + 1, 0)
            return _

        lax.fori_loop(0, (steps - 2) // 2, pair, 0)

        consume(steps - 1, 1)
        for p in range(2):
            for b in range(B):
                wait_chunk(p, b, sem_o)

    return sc_add(x, tab)


# speedup vs baseline: 1.0375x; 1.0144x over previous
import functools

import jax
import jax.numpy as jnp
from jax import lax
from jax.experimental import pallas as pl
from jax.experimental.pallas import tpu as pltpu
from jax.experimental.pallas import tpu_sc as plsc

_NC, _NS = 2, 16
_NW = _NC * _NS
_LANES = 16
_K = 8


def kernel(x, pos_table):
    B, L, D = x.shape
    tab = pos_table[:L]

    pw = L // _NW
    steps = pw // _K
    nvec = D // _LANES

    mesh = plsc.VectorSubcoreMesh(core_axis_name="c", subcore_axis_name="s")

    @functools.partial(
        pl.kernel,
        out_type=jax.ShapeDtypeStruct((B, L, D), jnp.float32),
        mesh=mesh,
        scratch_types=(
            [
                pltpu.VMEM((2, _K, D), jnp.float32),
                pltpu.VMEM((2, B, _K, D), jnp.float32),
            ]
            + [pltpu.SemaphoreType.DMA] * 2
            + [pltpu.SemaphoreType.DMA] * 2
            + [pltpu.SemaphoreType.DMA] * 2
        ),
    )
    def sc_add(x_hbm, tab_hbm, out_hbm, tbuf, obuf, *sems):
        sem_t = sems[0:2]
        sem_x = sems[2:4]
        sem_o = sems[4:6]

        wid = lax.axis_index("s") * _NC + lax.axis_index("c")
        pos0 = wid * pw

        def issue_ins(j, p):
            prow = pos0 + j * _K
            pltpu.async_copy(tab_hbm.at[pl.ds(prow, _K), :],
                             tbuf.at[p], sem_t[p])
            pltpu.async_copy(x_hbm.at[:, pl.ds(prow, _K), :],
                             obuf.at[p], sem_x[p])

        def wait_tab(p):
            pltpu.make_async_copy(tab_hbm.at[pl.ds(0, _K), :],
                                  tbuf.at[p], sem_t[p]).wait()

        def wait_x(p, sem2):
            pltpu.make_async_copy(x_hbm.at[:, pl.ds(0, _K), :],
                                  obuf.at[p], sem2[p]).wait()

        def compute(p):
            @plsc.parallel_loop(0, nvec, unroll=2)
            def _(i):
                off = i * _LANES
                for row in range(_K):
                    v = tbuf[p, row, pl.ds(off, _LANES)]
                    for b in range(B):
                        plsc.addupdate(
                            obuf.at[p, b, row, pl.ds(off, _LANES)], v)

        def consume(j, p):
            prow = pos0 + j * _K
            wait_tab(p)
            wait_x(p, sem_x)
            compute(p)
            pltpu.async_copy(obuf.at[p],
                             out_hbm.at[:, pl.ds(prow, _K), :], sem_o[p])

        def sub(j, p):
            wait_x(1 - p, sem_o)
            issue_ins(j + 1, 1 - p)
            consume(j, p)

        issue_ins(0, 0)
        issue_ins(1, 1)
        consume(0, 0)

        def pair(j2, _):
            j = 1 + j2 * 2
            sub(j, 1)
            sub(j + 1, 0)
            return _

        lax.fori_loop(0, (steps - 2) // 2, pair, 0)

        consume(steps - 1, 1)
        for p in range(2):
            wait_x(p, sem_o)

    return sc_add(x, tab)
